# baseline (speedup 1.0000x reference)
import functools

import jax
import jax.numpy as jnp
from jax import lax
from jax.experimental import pallas as pl
from jax.experimental.pallas import tpu as pltpu
from jax.experimental.pallas import tpu_sc as plsc

EMBED = 64
NC = 2
NS = 16
NW = NC * NS
CHUNK = 128


@functools.lru_cache(maxsize=None)
def _build(B, T):
    rows_per_w = B // NW
    mesh = plsc.VectorSubcoreMesh(core_axis_name="c", subcore_axis_name="s")

    splits = []
    off = 0
    while off < T:
        size = min(CHUNK, T - off)
        splits.append((off, size))
        off += size

    @functools.partial(
        pl.kernel,
        out_type=jax.ShapeDtypeStruct((B, T, EMBED), jnp.float32),
        mesh=mesh,
        compiler_params=pltpu.CompilerParams(use_tc_tiling_on_sc=False),
        scratch_types=[
            pltpu.VMEM((rows_per_w, T), jnp.int32),
            pltpu.VMEM((T, EMBED), jnp.float32),
            pltpu.VMEM((T, EMBED), jnp.float32),
            pltpu.SemaphoreType.DMA,
            pltpu.SemaphoreType.DMA,
            pltpu.SemaphoreType.DMA,
        ],
    )
    def emb(x_hbm, table_hbm, out_hbm, idx_v, rows0, rows1, gsem0, gsem1, osem):
        wid = lax.axis_index("s") * NC + lax.axis_index("c")
        b0 = wid * rows_per_w
        pltpu.sync_copy(x_hbm.at[pl.ds(b0, rows_per_w)], idx_v)

        bufs = (rows0, rows1)
        gsems = (gsem0, gsem1)

        def fire_gather(r, b):
            for off, size in splits:
                pltpu.async_copy(
                    table_hbm.at[idx_v.at[r, pl.ds(off, size)]],
                    bufs[b].at[pl.ds(off, size)],
                    gsems[b],
                )

        def drain_gather(b):
            pltpu.make_async_copy(
                table_hbm.at[pl.ds(0, T)], bufs[b], gsems[b]
            ).wait()

        fire_gather(0, 0)
        fire_gather(1, 1)

        @pl.loop(0, rows_per_w // 2)
        def _(p):
            for b in range(2):
                r = p * 2 + b
                drain_gather(b)
                pltpu.async_copy(bufs[b], out_hbm.at[b0 + r], osem).wait()

                @pl.when(r + 2 < rows_per_w)
                def _():
                    fire_gather(r + 2, b)

    return emb


def kernel(x, table):
    B, T = x.shape
    return _build(B, T)(x.astype(jnp.int32), table)
